# Initial kernel scaffold; baseline (speedup 1.0000x reference)
import functools

import jax
import jax.numpy as jnp
from jax import lax
from jax.experimental import pallas as pl
from jax.experimental.pallas import tpu as pltpu
from jax.experimental.pallas import tpu_sc as plsc

_INFO = plsc.get_sparse_core_info()
_NC, _NS, _L = _INFO.num_cores, _INFO.num_subcores, _INFO.num_lanes
_NW = _NC * _NS

_IDX_ROW = 128
_CR = 4
_CHUNK = _CR * _IDX_ROW


@functools.partial(jax.jit, static_argnums=(2, 3))
def _sc_gather(idx2d, table, n_rows, d):
    rows_per_w = n_rows // _NW
    chunks = rows_per_w // _CR

    mesh = plsc.VectorSubcoreMesh(core_axis_name="c", subcore_axis_name="s")

    @functools.partial(
        pl.kernel,
        out_type=jax.ShapeDtypeStruct((n_rows * _IDX_ROW, d), jnp.float32),
        mesh=mesh,
        scratch_types=[
            pltpu.VMEM((_CR, _IDX_ROW), jnp.int32),
            pltpu.VMEM((_CHUNK, d), jnp.float32),
            pltpu.SemaphoreType.DMA,
        ],
    )
    def k(idx_hbm, table_hbm, out_hbm, idx_v, rows_v, sem):
        wid = lax.axis_index("c") * _NS + lax.axis_index("s")
        base_row = wid * rows_per_w

        def body(g, carry):
            row0 = base_row + g * _CR
            pltpu.sync_copy(idx_hbm.at[pl.ds(row0, _CR)], idx_v)
            handles = []
            for j in range(_CR):
                handles.append(
                    pltpu.async_copy(
                        table_hbm.at[idx_v.at[j]],
                        rows_v.at[pl.ds(j * _IDX_ROW, _IDX_ROW)],
                        sem,
                    )
                )
            for h in handles:
                h.wait()
            pltpu.sync_copy(rows_v, out_hbm.at[pl.ds(row0 * _IDX_ROW, _CHUNK)])
            return carry

        lax.fori_loop(0, chunks, body, 0)

    return k(idx2d, table)


def kernel(token_ids, weight):
    b, s = token_ids.shape
    n = b * s
    idx2d = token_ids.reshape(n // _IDX_ROW, _IDX_ROW).astype(jnp.int32)
    out = _sc_gather(idx2d, weight, n // _IDX_ROW, weight.shape[1])
    return out.reshape(b, s, weight.shape[1])


# speedup vs baseline: 1.7963x; 1.7963x over previous
import functools

import jax
import jax.numpy as jnp
from jax import lax
from jax.experimental import pallas as pl
from jax.experimental.pallas import tpu as pltpu
from jax.experimental.pallas import tpu_sc as plsc

_INFO = plsc.get_sparse_core_info()
_NC, _NS, _L = _INFO.num_cores, _INFO.num_subcores, _INFO.num_lanes
_NW = _NC * _NS

_IDX_ROW = 128
_CR = 4
_CHUNK = _CR * _IDX_ROW


@functools.partial(jax.jit, static_argnums=(2, 3))
def _sc_gather(idx2d, table, n_rows, d):
    rows_per_w = n_rows // _NW
    chunks = rows_per_w // _CR

    mesh = plsc.VectorSubcoreMesh(core_axis_name="c", subcore_axis_name="s")

    @functools.partial(
        pl.kernel,
        out_type=jax.ShapeDtypeStruct((n_rows * _IDX_ROW, d), jnp.float32),
        mesh=mesh,
        scratch_types=[
            pltpu.VMEM((_CR, _IDX_ROW), jnp.int32),
            pltpu.VMEM((_CHUNK, d), jnp.float32),
            pltpu.SemaphoreType.DMA,
        ],
        compiler_params=pltpu.CompilerParams(use_tc_tiling_on_sc=False),
    )
    def k(idx_hbm, table_hbm, out_hbm, idx_v, rows_v, sem):
        wid = lax.axis_index("c") * _NS + lax.axis_index("s")
        base_row = wid * rows_per_w

        def body(g, carry):
            row0 = base_row + g * _CR
            pltpu.sync_copy(idx_hbm.at[pl.ds(row0, _CR)], idx_v)
            handles = []
            for j in range(_CR):
                handles.append(
                    pltpu.async_copy(
                        table_hbm.at[idx_v.at[j]],
                        rows_v.at[pl.ds(j * _IDX_ROW, _IDX_ROW)],
                        sem,
                    )
                )
            for h in handles:
                h.wait()
            pltpu.sync_copy(rows_v, out_hbm.at[pl.ds(row0 * _IDX_ROW, _CHUNK)])
            return carry

        lax.fori_loop(0, chunks, body, 0)

    return k(idx2d, table)


def kernel(token_ids, weight):
    b, s = token_ids.shape
    n = b * s
    idx2d = token_ids.reshape(n // _IDX_ROW, _IDX_ROW).astype(jnp.int32)
    out = _sc_gather(idx2d, weight, n // _IDX_ROW, weight.shape[1])
    return out.reshape(b, s, weight.shape[1])


# speedup vs baseline: 1.8763x; 1.0446x over previous
import functools

import jax
import jax.numpy as jnp
from jax import lax
from jax.experimental import pallas as pl
from jax.experimental.pallas import tpu as pltpu
from jax.experimental.pallas import tpu_sc as plsc

_INFO = plsc.get_sparse_core_info()
_NC, _NS, _L = _INFO.num_cores, _INFO.num_subcores, _INFO.num_lanes
_NW = _NC * _NS

_IDX_ROW = 128
_CR = 4
_CHUNK = _CR * _IDX_ROW


@functools.partial(jax.jit, static_argnums=(2, 3))
def _sc_gather(idx2d, table, n_rows, d):
    rows_per_w = n_rows // _NW
    chunks = rows_per_w // _CR
    assert chunks % 2 == 0

    mesh = plsc.VectorSubcoreMesh(core_axis_name="c", subcore_axis_name="s")

    @functools.partial(
        pl.kernel,
        out_type=jax.ShapeDtypeStruct((n_rows * _IDX_ROW, d), jnp.float32),
        mesh=mesh,
        scratch_types=[
            pltpu.VMEM((rows_per_w, _IDX_ROW), jnp.int32),
            pltpu.VMEM((2, _CHUNK, d), jnp.float32),
            pltpu.SemaphoreType.DMA,
            pltpu.SemaphoreType.DMA,
            pltpu.SemaphoreType.DMA,
            pltpu.SemaphoreType.DMA,
        ],
        compiler_params=pltpu.CompilerParams(use_tc_tiling_on_sc=False),
    )
    def k(idx_hbm, table_hbm, out_hbm, idx_v, rows_v, g0, g1, w0, w1):
        gsem = (g0, g1)
        wsem = (w0, w1)
        wid = lax.axis_index("c") * _NS + lax.axis_index("s")
        base_row = wid * rows_per_w

        pltpu.sync_copy(idx_hbm.at[pl.ds(base_row, rows_per_w)], idx_v)

        def fire_gather(g, b):
            for j in range(_CR):
                pltpu.async_copy(
                    table_hbm.at[idx_v.at[g * _CR + j]],
                    rows_v.at[b].at[pl.ds(j * _IDX_ROW, _IDX_ROW)],
                    gsem[b],
                )

        def wait_gather(b):
            pltpu.make_async_copy(
                out_hbm.at[pl.ds(0, _CHUNK)], rows_v.at[b], gsem[b]
            ).wait()

        def fire_writeback(g, b):
            pltpu.async_copy(
                rows_v.at[b],
                out_hbm.at[pl.ds((base_row + g * _CR) * _IDX_ROW, _CHUNK)],
                wsem[b],
            )

        def wait_writeback(b):
            pltpu.make_async_copy(
                rows_v.at[b], out_hbm.at[pl.ds(0, _CHUNK)], wsem[b]
            ).wait()

        fire_gather(0, 0)

        def body(i, carry):
            for b in range(2):
                g = i * 2 + b
                nb = 1 - b

                def _wait_prev_wb():
                    wait_writeback(nb)

                if b == 1:
                    _wait_prev_wb()
                else:
                    pl.when(g >= 1)(_wait_prev_wb)

                def _fire_next():
                    fire_gather(g + 1, nb)

                pl.when(g + 1 < chunks)(_fire_next)
                wait_gather(b)
                fire_writeback(g, b)
            return carry

        lax.fori_loop(0, chunks // 2, body, 0)
        wait_writeback((chunks - 1) % 2)

    return k(idx2d, table)


def kernel(token_ids, weight):
    b, s = token_ids.shape
    n = b * s
    idx2d = token_ids.reshape(n // _IDX_ROW, _IDX_ROW).astype(jnp.int32)
    out = _sc_gather(idx2d, weight, n // _IDX_ROW, weight.shape[1])
    return out.reshape(b, s, weight.shape[1])
